# baseline (speedup 1.0000x reference)
import functools

import numpy as np

import jax
import jax.numpy as jnp
from jax import lax
from jax.experimental import pallas as pl
from jax.experimental.pallas import tpu as pltpu, tpu_sc as plsc

N = 10000
V = 100000
D = 128
H = 128
L = 20
E = 320000
B = 4096
ALPHA = 0.2
NPAD = 10240
AW = 128

NC = 2
NS = 16
NW = NC * NS

_mesh = plsc.VectorSubcoreMesh(core_axis_name="c", subcore_axis_name="s")
_sc_params = pltpu.CompilerParams(needs_layout_passes=False)


_K1_G = 16
_K1_CHUNK = 320
_K1_NG = _K1_CHUNK // _K1_G


def _k1_body(fi_flat, we, out,
             ia0, ib0, ia1, ib1, rows0, rows1, out_v, semA, semB):
    c = lax.axis_index("c")
    s = lax.axis_index("s")
    w = s * NC + c
    wbase = jnp.minimum(w * _K1_CHUNK, N - _K1_CHUNK)

    def fetch(g, ia, ib, rows, sem):
        base = wbase + g * _K1_G
        pltpu.sync_copy(fi_flat.at[pl.ds(base * L, 160)], ia)
        pltpu.sync_copy(fi_flat.at[pl.ds(base * L + 160, 160)], ib)
        for p in range(2):
            pltpu.async_copy(we.at[ia.at[pl.ds(p * 80, 80)]],
                             rows.at[pl.ds(p * 80, 80)], sem)
            pltpu.async_copy(we.at[ib.at[pl.ds(p * 80, 80)]],
                             rows.at[pl.ds(160 + p * 80, 80)], sem)

    def drain(ia, ib, rows, sem):
        for p in range(2):
            pltpu.make_async_copy(we.at[ia.at[pl.ds(p * 80, 80)]],
                                  rows.at[pl.ds(p * 80, 80)], sem).wait()
            pltpu.make_async_copy(we.at[ib.at[pl.ds(p * 80, 80)]],
                                  rows.at[pl.ds(160 + p * 80, 80)], sem).wait()

    def consume(g, rows):
        def tweet(t, _):
            for j in range(8):
                acc = [rows[t * L + l, pl.ds(j * 16, 16)] for l in range(L)]
                while len(acc) > 1:
                    acc = [a + b for a, b in zip(acc[::2], acc[1::2])] + (
                        [acc[-1]] if len(acc) % 2 else [])
                out_v[g * _K1_G + t, pl.ds(j * 16, 16)] = acc[0] * (1.0 / L)
            return 0

        lax.fori_loop(0, _K1_G, tweet, 0)

    fetch(0, ia0, ib0, rows0, semA)

    def super_iter(t, _):
        g1 = 2 * t + 1
        fetch(g1, ia1, ib1, rows1, semB)
        drain(ia0, ib0, rows0, semA)
        consume(2 * t, rows0)

        @pl.when(t < _K1_NG // 2 - 1)
        def _():
            fetch(g1 + 1, ia0, ib0, rows0, semA)

        drain(ia1, ib1, rows1, semB)
        consume(g1, rows1)
        return 0

    lax.fori_loop(0, _K1_NG // 2, super_iter, 0)
    pltpu.sync_copy(out_v, out.at[pl.ds(wbase, _K1_CHUNK)])


_k1 = pl.kernel(
    _k1_body,
    out_type=jax.ShapeDtypeStruct((N, D), jnp.float32),
    mesh=_mesh,
    compiler_params=_sc_params,
    scratch_types=[
        pltpu.VMEM((160,), jnp.int32),
        pltpu.VMEM((160,), jnp.int32),
        pltpu.VMEM((160,), jnp.int32),
        pltpu.VMEM((160,), jnp.int32),
        pltpu.VMEM((_K1_G * L, D), jnp.float32),
        pltpu.VMEM((_K1_G * L, D), jnp.float32),
        pltpu.VMEM((_K1_CHUNK, D), jnp.float32),
        pltpu.SemaphoreType.DMA,
        pltpu.SemaphoreType.DMA,
    ],
)


def _tc_body(x_ref, w_ref, a2_ref, h_ref, sd_ref):
    h = jnp.dot(x_ref[...], w_ref[...], preferred_element_type=jnp.float32)
    h_ref[...] = h
    sd_ref[...] = jnp.dot(h, a2_ref[...], preferred_element_type=jnp.float32)


_TC_BLK = 400

_tc = pl.pallas_call(
    _tc_body,
    grid=(N // _TC_BLK,),
    in_specs=[
        pl.BlockSpec((_TC_BLK, D), lambda i: (i, 0)),
        pl.BlockSpec((D, H), lambda i: (0, 0)),
        pl.BlockSpec((H, 8), lambda i: (0, 0)),
    ],
    out_specs=[
        pl.BlockSpec((_TC_BLK, H), lambda i: (i, 0)),
        pl.BlockSpec((_TC_BLK, 8), lambda i: (i, 0)),
    ],
    out_shape=[
        jax.ShapeDtypeStruct((N, H), jnp.float32),
        jax.ShapeDtypeStruct((N, 8), jnp.float32),
    ],
)


_K2_C = 80
_K2_NCH = (E // NW) // _K2_C
_STRIPE = NPAD // NS


def _k2_body(pk_hbm, s_hbm, d_hbm, h_hbm, outA, outB, denA, denB,
             pk_all,
             src0, dst0, ee0, sg0, dg0, rows0,
             src1, dst1, ee1, sg1, dg1, rows1,
             rows_s, dsts,
             acc_sh, den_sh, s_sh, d_sh,
             semG0, semG1, semS):
    c = lax.axis_index("c")
    sid = lax.axis_index("s")
    w = sid * NC + c
    z16 = jnp.zeros((16,), jnp.float32)

    def zrow(r, _):
        for j in range(8):
            rows0[r, pl.ds(j * 16, 16)] = z16
        return 0

    lax.fori_loop(0, _K2_C, zrow, 0)
    for k in range(_K2_C // 16):
        ee0[pl.ds(k * 16, 16)] = z16
    r0 = sid * _STRIPE

    def zcp(k, _):
        pltpu.sync_copy(rows0, acc_sh.at[pl.ds(r0 + k * _K2_C, _K2_C)])
        return 0

    lax.fori_loop(0, _STRIPE // _K2_C, zcp, 0)
    for k in range(_STRIPE // _K2_C):
        pltpu.sync_copy(ee0, den_sh.at[pl.ds(r0 + k * _K2_C, _K2_C)])

    @pl.when(sid == 0)
    def _():
        pltpu.sync_copy(s_hbm, s_sh)
        pltpu.sync_copy(d_hbm, d_sh)

    e_base = w * (E // NW)
    pltpu.sync_copy(pk_hbm.at[pl.ds(e_base, E // NW)], pk_all)
    plsc.subcore_barrier()

    def prep(cc, src_b, dst_b, ee_b, sg_b, dg_b):
        for k in range(_K2_C // 16):
            pk = pk_all[pl.ds(cc * _K2_C + k * 16, 16)]
            src_b[pl.ds(k * 16, 16)] = jnp.bitwise_and(pk, 16383)
            dst_b[pl.ds(k * 16, 16)] = lax.shift_right_logical(pk, 14)
        pltpu.sync_copy(s_sh.at[src_b], sg_b)
        pltpu.sync_copy(d_sh.at[dst_b], dg_b)
        for k in range(_K2_C // 16):
            e = sg_b[pl.ds(k * 16, 16)] + dg_b[pl.ds(k * 16, 16)]
            e = jnp.where(e > 0, e, e * ALPHA)
            ee = jnp.exp(e)
            ee_b[pl.ds(k * 16, 16)] = ee
        pltpu.sync_copy(ee_b, den_sh.at[dst_b], add=True)

    def fin(rows_g, dst_b, ee_b):
        for k in range(_K2_C // 16):
            dsts[pl.ds(k * 16, 16)] = dst_b[pl.ds(k * 16, 16)]

        def edge_i(i, _):
            eb = plsc.load_gather(ee_b, [jnp.full((16,), 0, jnp.int32) + i])
            for j in range(8):
                rows_s[i, pl.ds(j * 16, 16)] = rows_g[i, pl.ds(j * 16, 16)] * eb
            return 0

        lax.fori_loop(0, _K2_C, edge_i, 0)
        pltpu.async_copy(rows_s, acc_sh.at[dsts], semS, add=True)

    def wait_s():
        pltpu.make_async_copy(rows_s, acc_sh.at[dsts], semS).wait()

    def gwait(src_b, rows_g, semG):
        pltpu.make_async_copy(h_hbm.at[src_b], rows_g, semG).wait()

    prep(0, src0, dst0, ee0, sg0, dg0)
    pltpu.async_copy(h_hbm.at[src0], rows0, semG0)
    prep(1, src1, dst1, ee1, sg1, dg1)
    pltpu.async_copy(h_hbm.at[src1], rows1, semG1)
    nsup = (_K2_NCH - 1) // 2

    def super_iter(t, _):
        gwait(src0, rows0, semG0)

        @pl.when(t > 0)
        def _():
            wait_s()

        fin(rows0, dst0, ee0)
        prep(2 * t + 2, src0, dst0, ee0, sg0, dg0)
        pltpu.async_copy(h_hbm.at[src0], rows0, semG0)

        gwait(src1, rows1, semG1)
        wait_s()
        fin(rows1, dst1, ee1)

        @pl.when(t < nsup - 1)
        def _():
            prep(2 * t + 3, src1, dst1, ee1, sg1, dg1)
            pltpu.async_copy(h_hbm.at[src1], rows1, semG1)

        return 0

    lax.fori_loop(0, nsup, super_iter, 0)
    gwait(src0, rows0, semG0)
    wait_s()
    fin(rows0, dst0, ee0)
    wait_s()
    plsc.subcore_barrier()

    @pl.when(c == 0)
    def _():
        pltpu.sync_copy(acc_sh.at[pl.ds(r0, _STRIPE)], outA.at[pl.ds(r0, _STRIPE)])
        pltpu.sync_copy(den_sh.at[pl.ds(r0, _STRIPE)], denA.at[pl.ds(r0, _STRIPE)])

    @pl.when(c == 1)
    def _():
        pltpu.sync_copy(acc_sh.at[pl.ds(r0, _STRIPE)], outB.at[pl.ds(r0, _STRIPE)])
        pltpu.sync_copy(den_sh.at[pl.ds(r0, _STRIPE)], denB.at[pl.ds(r0, _STRIPE)])


def _edge_bufs():
    return [
        pltpu.VMEM((_K2_C,), jnp.int32),
        pltpu.VMEM((_K2_C,), jnp.int32),
        pltpu.VMEM((_K2_C,), jnp.float32),
        pltpu.VMEM((_K2_C,), jnp.float32),
        pltpu.VMEM((_K2_C,), jnp.float32),
        pltpu.VMEM((_K2_C, H), jnp.float32),
    ]


_k2 = pl.kernel(
    _k2_body,
    out_type=(
        jax.ShapeDtypeStruct((NPAD, AW), jnp.float32),
        jax.ShapeDtypeStruct((NPAD, AW), jnp.float32),
        jax.ShapeDtypeStruct((NPAD,), jnp.float32),
        jax.ShapeDtypeStruct((NPAD,), jnp.float32),
    ),
    mesh=_mesh,
    compiler_params=_sc_params,
    scratch_types=[
        pltpu.VMEM((E // NW,), jnp.int32),
        *_edge_bufs(),
        *_edge_bufs(),
        pltpu.VMEM((_K2_C, H), jnp.float32),
        pltpu.VMEM((_K2_C,), jnp.int32),
        pltpu.VMEM_SHARED((NPAD, AW), jnp.float32),
        pltpu.VMEM_SHARED((NPAD,), jnp.float32),
        pltpu.VMEM_SHARED((N,), jnp.float32),
        pltpu.VMEM_SHARED((N,), jnp.float32),
        pltpu.SemaphoreType.DMA,
        pltpu.SemaphoreType.DMA,
        pltpu.SemaphoreType.DMA,
    ],
)


_K4_Q = B // NW


def _k4_body(tw_hbm, accA, accB, denA, denB, out,
             idx_v, rowsA, rowsB, dA_v, dB_v, r_v, out_v, sem):
    c = lax.axis_index("c")
    sid = lax.axis_index("s")
    w = sid * NC + c
    q0 = w * _K4_Q
    pltpu.sync_copy(tw_hbm.at[pl.ds(q0, _K4_Q)], idx_v)
    cpa = pltpu.async_copy(accA.at[idx_v], rowsA, sem)
    cpb = pltpu.async_copy(accB.at[idx_v], rowsB, sem)
    pltpu.sync_copy(denA, dA_v)
    pltpu.sync_copy(denB, dB_v)
    for k in range(_K4_Q // 16):
        qi = idx_v[pl.ds(k * 16, 16)]
        da = plsc.load_gather(dA_v, [qi])
        db = plsc.load_gather(dB_v, [qi])
        r_v[pl.ds(k * 16, 16)] = 1.0 / (da + db + 1e-16)
    cpa.wait()
    cpb.wait()

    def q(i, _):
        rb = plsc.load_gather(r_v, [jnp.full((16,), 0, jnp.int32) + i])
        for j in range(8):
            o = (rowsA[i, pl.ds(j * 16, 16)] + rowsB[i, pl.ds(j * 16, 16)]) * rb
            out_v[i, pl.ds(j * 16, 16)] = jnp.where(o > 0, o, jnp.exp(o) - 1.0)
        return 0

    lax.fori_loop(0, _K4_Q, q, 0)
    pltpu.sync_copy(out_v, out.at[pl.ds(q0, _K4_Q)])


_k4 = pl.kernel(
    _k4_body,
    out_type=jax.ShapeDtypeStruct((B, H), jnp.float32),
    mesh=_mesh,
    compiler_params=_sc_params,
    scratch_types=[
        pltpu.VMEM((_K4_Q,), jnp.int32),
        pltpu.VMEM((_K4_Q, AW), jnp.float32),
        pltpu.VMEM((_K4_Q, AW), jnp.float32),
        pltpu.VMEM((NPAD,), jnp.float32),
        pltpu.VMEM((NPAD,), jnp.float32),
        pltpu.VMEM((_K4_Q,), jnp.float32),
        pltpu.VMEM((_K4_Q, H), jnp.float32),
        pltpu.SemaphoreType.DMA,
    ],
)


@jax.jit
def kernel(word_embedding, features_index, edge_index, W, a_src, a_dst,
           tw_graph_idx, ut_graph_idx):
    fi_flat = features_index.reshape(-1)
    twt = _k1(fi_flat, word_embedding)
    a2 = jnp.zeros((H, 8), jnp.float32)
    a2 = a2.at[:, 0].set(a_src).at[:, 1].set(a_dst)
    h, sd = _tc(twt, W, a2)
    s = sd[:, 0]
    d = sd[:, 1]
    pk = jnp.bitwise_or(jnp.left_shift(edge_index[1], 14), edge_index[0])
    accA, accB, denA, denB = _k2(pk, s, d, h)
    return _k4(tw_graph_idx, accA, accB, denA, denB)


# speedup vs baseline: 1.6320x; 1.6320x over previous
import functools

import numpy as np

import jax
import jax.numpy as jnp
from jax import lax
from jax.experimental import pallas as pl
from jax.experimental.pallas import tpu as pltpu, tpu_sc as plsc

N = 10000
V = 100000
D = 128
H = 128
L = 20
E = 320000
B = 4096
ALPHA = 0.2
NPAD = 10240
AW = 128

NC = 2
NS = 16
NW = NC * NS

_mesh = plsc.VectorSubcoreMesh(core_axis_name="c", subcore_axis_name="s")
_sc_params = pltpu.CompilerParams(needs_layout_passes=False)


_K1_G = 16
_K1_CHUNK = 320
_K1_NG = _K1_CHUNK // _K1_G


def _k1_body(fi_flat, we, out,
             ia0, ib0, ia1, ib1, rows0, rows1, out_v, semA, semB):
    c = lax.axis_index("c")
    s = lax.axis_index("s")
    w = s * NC + c
    wbase = jnp.minimum(w * _K1_CHUNK, N - _K1_CHUNK)

    def fetch(g, ia, ib, rows, sem):
        base = wbase + g * _K1_G
        pltpu.sync_copy(fi_flat.at[pl.ds(base * L, 160)], ia)
        pltpu.sync_copy(fi_flat.at[pl.ds(base * L + 160, 160)], ib)
        for p in range(2):
            pltpu.async_copy(we.at[ia.at[pl.ds(p * 80, 80)]],
                             rows.at[pl.ds(p * 80, 80)], sem)
            pltpu.async_copy(we.at[ib.at[pl.ds(p * 80, 80)]],
                             rows.at[pl.ds(160 + p * 80, 80)], sem)

    def drain(ia, ib, rows, sem):
        for p in range(2):
            pltpu.make_async_copy(we.at[ia.at[pl.ds(p * 80, 80)]],
                                  rows.at[pl.ds(p * 80, 80)], sem).wait()
            pltpu.make_async_copy(we.at[ib.at[pl.ds(p * 80, 80)]],
                                  rows.at[pl.ds(160 + p * 80, 80)], sem).wait()

    def consume(g, rows):
        def tweet(t, _):
            for j in range(8):
                acc = [rows[t * L + l, pl.ds(j * 16, 16)] for l in range(L)]
                while len(acc) > 1:
                    acc = [a + b for a, b in zip(acc[::2], acc[1::2])] + (
                        [acc[-1]] if len(acc) % 2 else [])
                out_v[g * _K1_G + t, pl.ds(j * 16, 16)] = acc[0] * (1.0 / L)
            return 0

        lax.fori_loop(0, _K1_G, tweet, 0)

    fetch(0, ia0, ib0, rows0, semA)

    def super_iter(t, _):
        g1 = 2 * t + 1
        fetch(g1, ia1, ib1, rows1, semB)
        drain(ia0, ib0, rows0, semA)
        consume(2 * t, rows0)

        @pl.when(t < _K1_NG // 2 - 1)
        def _():
            fetch(g1 + 1, ia0, ib0, rows0, semA)

        drain(ia1, ib1, rows1, semB)
        consume(g1, rows1)
        return 0

    lax.fori_loop(0, _K1_NG // 2, super_iter, 0)
    pltpu.sync_copy(out_v, out.at[pl.ds(wbase, _K1_CHUNK)])


_k1 = pl.kernel(
    _k1_body,
    out_type=jax.ShapeDtypeStruct((N, D), jnp.float32),
    mesh=_mesh,
    compiler_params=_sc_params,
    scratch_types=[
        pltpu.VMEM((160,), jnp.int32),
        pltpu.VMEM((160,), jnp.int32),
        pltpu.VMEM((160,), jnp.int32),
        pltpu.VMEM((160,), jnp.int32),
        pltpu.VMEM((_K1_G * L, D), jnp.float32),
        pltpu.VMEM((_K1_G * L, D), jnp.float32),
        pltpu.VMEM((_K1_CHUNK, D), jnp.float32),
        pltpu.SemaphoreType.DMA,
        pltpu.SemaphoreType.DMA,
    ],
)


def _tc_body(x_ref, w_ref, a2_ref, h_ref, sd_ref):
    h = jnp.dot(x_ref[...], w_ref[...], preferred_element_type=jnp.float32)
    h_ref[...] = h
    sd_ref[...] = jnp.dot(h, a2_ref[...], preferred_element_type=jnp.float32)


_TC_BLK = 400

_tc = pl.pallas_call(
    _tc_body,
    grid=(N // _TC_BLK,),
    in_specs=[
        pl.BlockSpec((_TC_BLK, D), lambda i: (i, 0)),
        pl.BlockSpec((D, H), lambda i: (0, 0)),
        pl.BlockSpec((H, 8), lambda i: (0, 0)),
    ],
    out_specs=[
        pl.BlockSpec((_TC_BLK, H), lambda i: (i, 0)),
        pl.BlockSpec((_TC_BLK, 8), lambda i: (i, 0)),
    ],
    out_shape=[
        jax.ShapeDtypeStruct((N, H), jnp.float32),
        jax.ShapeDtypeStruct((N, 8), jnp.float32),
    ],
)


_K2_C = 80
_K2_NCH = (E // NW) // _K2_C
_STRIPE = NPAD // NS


def _k2_body(pk_hbm, s_hbm, d_hbm, h_hbm, outA, outB, denA, denB,
             pk_all,
             src0, dst0, ee0, sg0, dg0, rows0,
             src1, dst1, ee1, sg1, dg1, rows1,
             acc_sh, den_sh, s_sh, d_sh,
             semG0, semG1, semS0, semS1):
    c = lax.axis_index("c")
    sid = lax.axis_index("s")
    w = sid * NC + c
    z16 = jnp.zeros((16,), jnp.float32)

    def zrow(r, _):
        for j in range(8):
            rows0[r, pl.ds(j * 16, 16)] = z16
        return 0

    lax.fori_loop(0, _K2_C, zrow, 0)
    for k in range(_K2_C // 16):
        ee0[pl.ds(k * 16, 16)] = z16
    r0 = sid * _STRIPE

    def zcp(k, _):
        pltpu.sync_copy(rows0, acc_sh.at[pl.ds(r0 + k * _K2_C, _K2_C)])
        return 0

    lax.fori_loop(0, _STRIPE // _K2_C, zcp, 0)
    for k in range(_STRIPE // _K2_C):
        pltpu.sync_copy(ee0, den_sh.at[pl.ds(r0 + k * _K2_C, _K2_C)])

    @pl.when(sid == 0)
    def _():
        pltpu.sync_copy(s_hbm, s_sh)
        pltpu.sync_copy(d_hbm, d_sh)

    e_base = w * (E // NW)
    pltpu.sync_copy(pk_hbm.at[pl.ds(e_base, E // NW)], pk_all)
    plsc.subcore_barrier()

    def prep(cc, src_b, dst_b, ee_b, sg_b, dg_b):
        for k in range(_K2_C // 16):
            pk = pk_all[pl.ds(cc * _K2_C + k * 16, 16)]
            src_b[pl.ds(k * 16, 16)] = jnp.bitwise_and(pk, 16383)
            dst_b[pl.ds(k * 16, 16)] = lax.shift_right_logical(pk, 14)
        pltpu.sync_copy(s_sh.at[src_b], sg_b)
        pltpu.sync_copy(d_sh.at[dst_b], dg_b)
        for k in range(_K2_C // 16):
            e = sg_b[pl.ds(k * 16, 16)] + dg_b[pl.ds(k * 16, 16)]
            e = jnp.where(e > 0, e, e * ALPHA)
            ee = jnp.exp(e)
            ee_b[pl.ds(k * 16, 16)] = ee
        pltpu.sync_copy(ee_b, den_sh.at[dst_b], add=True)

    def fin(rows_b, dst_b, ee_b, semS):
        def edge_i(i, _):
            eb = plsc.load_gather(ee_b, [jnp.full((16,), 0, jnp.int32) + i])
            for j in range(8):
                rows_b[i, pl.ds(j * 16, 16)] = rows_b[i, pl.ds(j * 16, 16)] * eb
            return 0

        lax.fori_loop(0, _K2_C, edge_i, 0)
        pltpu.async_copy(rows_b, acc_sh.at[dst_b], semS, add=True)

    def wait_s(rows_b, dst_b, semS):
        pltpu.make_async_copy(rows_b, acc_sh.at[dst_b], semS).wait()

    def gwait(src_b, rows_b, semG):
        pltpu.make_async_copy(h_hbm.at[src_b], rows_b, semG).wait()

    prep(0, src0, dst0, ee0, sg0, dg0)
    pltpu.async_copy(h_hbm.at[src0], rows0, semG0)
    nsup = (_K2_NCH - 1) // 2

    def super_iter(t, _):
        @pl.when(t > 0)
        def _():
            wait_s(rows1, dst1, semS1)

        prep(2 * t + 1, src1, dst1, ee1, sg1, dg1)
        pltpu.async_copy(h_hbm.at[src1], rows1, semG1)
        gwait(src0, rows0, semG0)
        fin(rows0, dst0, ee0, semS0)

        wait_s(rows0, dst0, semS0)
        prep(2 * t + 2, src0, dst0, ee0, sg0, dg0)
        pltpu.async_copy(h_hbm.at[src0], rows0, semG0)
        gwait(src1, rows1, semG1)
        fin(rows1, dst1, ee1, semS1)
        return 0

    lax.fori_loop(0, nsup, super_iter, 0)
    gwait(src0, rows0, semG0)
    wait_s(rows1, dst1, semS1)
    fin(rows0, dst0, ee0, semS0)
    wait_s(rows0, dst0, semS0)
    plsc.subcore_barrier()

    @pl.when(c == 0)
    def _():
        pltpu.sync_copy(acc_sh.at[pl.ds(r0, _STRIPE)], outA.at[pl.ds(r0, _STRIPE)])
        pltpu.sync_copy(den_sh.at[pl.ds(r0, _STRIPE)], denA.at[pl.ds(r0, _STRIPE)])

    @pl.when(c == 1)
    def _():
        pltpu.sync_copy(acc_sh.at[pl.ds(r0, _STRIPE)], outB.at[pl.ds(r0, _STRIPE)])
        pltpu.sync_copy(den_sh.at[pl.ds(r0, _STRIPE)], denB.at[pl.ds(r0, _STRIPE)])


def _edge_bufs():
    return [
        pltpu.VMEM((_K2_C,), jnp.int32),
        pltpu.VMEM((_K2_C,), jnp.int32),
        pltpu.VMEM((_K2_C,), jnp.float32),
        pltpu.VMEM((_K2_C,), jnp.float32),
        pltpu.VMEM((_K2_C,), jnp.float32),
        pltpu.VMEM((_K2_C, H), jnp.float32),
    ]


_k2 = pl.kernel(
    _k2_body,
    out_type=(
        jax.ShapeDtypeStruct((NPAD, AW), jnp.float32),
        jax.ShapeDtypeStruct((NPAD, AW), jnp.float32),
        jax.ShapeDtypeStruct((NPAD,), jnp.float32),
        jax.ShapeDtypeStruct((NPAD,), jnp.float32),
    ),
    mesh=_mesh,
    compiler_params=_sc_params,
    scratch_types=[
        pltpu.VMEM((E // NW,), jnp.int32),
        *_edge_bufs(),
        *_edge_bufs(),
        pltpu.VMEM_SHARED((NPAD, AW), jnp.float32),
        pltpu.VMEM_SHARED((NPAD,), jnp.float32),
        pltpu.VMEM_SHARED((N,), jnp.float32),
        pltpu.VMEM_SHARED((N,), jnp.float32),
        pltpu.SemaphoreType.DMA,
        pltpu.SemaphoreType.DMA,
        pltpu.SemaphoreType.DMA,
        pltpu.SemaphoreType.DMA,
    ],
)


_K4_Q = B // NW


def _k4_body(tw_hbm, accA, accB, denA, denB, out,
             idx_v, rowsA, rowsB, dA_v, dB_v, r_v, out_v, sem):
    c = lax.axis_index("c")
    sid = lax.axis_index("s")
    w = sid * NC + c
    q0 = w * _K4_Q
    pltpu.sync_copy(tw_hbm.at[pl.ds(q0, _K4_Q)], idx_v)
    cpa = pltpu.async_copy(accA.at[idx_v], rowsA, sem)
    cpb = pltpu.async_copy(accB.at[idx_v], rowsB, sem)
    pltpu.sync_copy(denA, dA_v)
    pltpu.sync_copy(denB, dB_v)
    for k in range(_K4_Q // 16):
        qi = idx_v[pl.ds(k * 16, 16)]
        da = plsc.load_gather(dA_v, [qi])
        db = plsc.load_gather(dB_v, [qi])
        r_v[pl.ds(k * 16, 16)] = 1.0 / (da + db + 1e-16)
    cpa.wait()
    cpb.wait()

    def q(i, _):
        rb = plsc.load_gather(r_v, [jnp.full((16,), 0, jnp.int32) + i])
        for j in range(8):
            o = (rowsA[i, pl.ds(j * 16, 16)] + rowsB[i, pl.ds(j * 16, 16)]) * rb
            out_v[i, pl.ds(j * 16, 16)] = jnp.where(o > 0, o, jnp.exp(o) - 1.0)
        return 0

    lax.fori_loop(0, _K4_Q, q, 0)
    pltpu.sync_copy(out_v, out.at[pl.ds(q0, _K4_Q)])


_k4 = pl.kernel(
    _k4_body,
    out_type=jax.ShapeDtypeStruct((B, H), jnp.float32),
    mesh=_mesh,
    compiler_params=_sc_params,
    scratch_types=[
        pltpu.VMEM((_K4_Q,), jnp.int32),
        pltpu.VMEM((_K4_Q, AW), jnp.float32),
        pltpu.VMEM((_K4_Q, AW), jnp.float32),
        pltpu.VMEM((NPAD,), jnp.float32),
        pltpu.VMEM((NPAD,), jnp.float32),
        pltpu.VMEM((_K4_Q,), jnp.float32),
        pltpu.VMEM((_K4_Q, H), jnp.float32),
        pltpu.SemaphoreType.DMA,
    ],
)


@jax.jit
def kernel(word_embedding, features_index, edge_index, W, a_src, a_dst,
           tw_graph_idx, ut_graph_idx):
    fi_flat = features_index.reshape(-1)
    twt = _k1(fi_flat, word_embedding)
    a2 = jnp.zeros((H, 8), jnp.float32)
    a2 = a2.at[:, 0].set(a_src).at[:, 1].set(a_dst)
    h, sd = _tc(twt, W, a2)
    s = sd[:, 0]
    d = sd[:, 1]
    pk = jnp.bitwise_or(jnp.left_shift(edge_index[1], 14), edge_index[0])
    accA, accB, denA, denB = _k2(pk, s, d, h)
    return _k4(tw_graph_idx, accA, accB, denA, denB)
